# baseline (speedup 1.0000x reference)
import functools

import jax
import jax.numpy as jnp
from jax import lax
from jax.experimental import pallas as pl
from jax.experimental.pallas import tpu as pltpu
from jax.experimental.pallas import tpu_sc as plsc

N_NODES = 10000
N_EDGES = 320000
IN_FEATS = 128
H_FEATS = 128
NUM_CLASSES = 64

NPAD = 10240
DUMMY = NPAD - 1
NC = 2
NS = 16
NW = NC * NS
CH = 128
NCH = 79
EPAD = NW * NCH * CH
ROWS_PER_TILE = NPAD // NS
HF = 64
DEG_W = 16

_mesh = plsc.VectorSubcoreMesh(core_axis_name="c", subcore_axis_name="s")
_sc_params = pltpu.CompilerParams(use_tc_tiling_on_sc=False)


def _zero_fill(ref, nrows, width):
    zv = jnp.zeros((16,), jnp.float32)

    def body(i, _):
        for k in range(width // 16):
            ref[i, pl.ds(k * 16, 16)] = zv
        return 0

    lax.fori_loop(0, nrows, body, 0, unroll=False)


def _ones_fill(ref, nrows, width):
    ov = jnp.ones((16,), jnp.float32)

    def body(i, _):
        for k in range(width // 16):
            ref[i, pl.ds(k * 16, 16)] = ov
        return 0

    lax.fori_loop(0, nrows, body, 0, unroll=False)


@functools.partial(
    pl.kernel,
    out_type=(
        jax.ShapeDtypeStruct((NC, NPAD, DEG_W), jnp.float32),
        jax.ShapeDtypeStruct((NC, NPAD, DEG_W), jnp.float32),
    ),
    mesh=_mesh,
    compiler_params=_sc_params,
    scratch_types=[
        pltpu.VMEM((NCH, CH), jnp.int32),
        pltpu.VMEM((NCH, CH), jnp.int32),
        pltpu.VMEM((CH, DEG_W), jnp.float32),
        pltpu.VMEM_SHARED((NPAD, DEG_W), jnp.float32),
        pltpu.VMEM_SHARED((NPAD, DEG_W), jnp.float32),
        pltpu.SemaphoreType.DMA,
        pltpu.SemaphoreType.DMA,
    ],
)
def _deg_kernel(src_hbm, dst_hbm, outs_hbm, outd_hbm, sidx, didx, buf, accs,
                accd, sems, semd):
    c = lax.axis_index("c")
    s = lax.axis_index("s")
    w = s * NC + c
    base = s * ROWS_PER_TILE
    _zero_fill(buf, CH, DEG_W)
    for r in range(ROWS_PER_TILE // CH):
        pltpu.sync_copy(buf, accs.at[pl.ds(base + r * CH, CH)])
        pltpu.sync_copy(buf, accd.at[pl.ds(base + r * CH, CH)])
    _ones_fill(buf, CH, DEG_W)
    plsc.subcore_barrier()

    pltpu.sync_copy(src_hbm.at[w], sidx)
    pltpu.sync_copy(dst_hbm.at[w], didx)

    pltpu.async_copy(buf, accs.at[sidx.at[0]], sems, add=True)
    pltpu.async_copy(buf, accd.at[didx.at[0]], semd, add=True)

    def body(j, _):
        pltpu.make_async_copy(buf, accs.at[sidx.at[j - 1]], sems).wait()
        pltpu.make_async_copy(buf, accd.at[didx.at[j - 1]], semd).wait()
        pltpu.async_copy(buf, accs.at[sidx.at[j]], sems, add=True)
        pltpu.async_copy(buf, accd.at[didx.at[j]], semd, add=True)
        return 0

    lax.fori_loop(1, NCH, body, 0, unroll=False)
    pltpu.make_async_copy(buf, accs.at[sidx.at[NCH - 1]], sems).wait()
    pltpu.make_async_copy(buf, accd.at[didx.at[NCH - 1]], semd).wait()
    plsc.subcore_barrier()

    pltpu.sync_copy(accs.at[pl.ds(base, ROWS_PER_TILE)],
                    outs_hbm.at[c, pl.ds(base, ROWS_PER_TILE)])
    pltpu.sync_copy(accd.at[pl.ds(base, ROWS_PER_TILE)],
                    outd_hbm.at[c, pl.ds(base, ROWS_PER_TILE)])


def _zero_acc(zbuf, acc, base, zsem):
    _zero_fill(zbuf, CH, HF)
    ds = [pltpu.async_copy(zbuf, acc.at[pl.ds(base + r * CH, CH)], zsem)
          for r in range(ROWS_PER_TILE // CH)]
    for d in ds:
        d.wait()


def _ring_loop(h_hbm, acc, sidx, didx, rows, gsems, ssems, nch):
    pltpu.async_copy(h_hbm.at[sidx.at[0]], rows[0], gsems[0])
    pltpu.async_copy(h_hbm.at[sidx.at[1]], rows[1], gsems[1])

    def group(q, _):
        for k in range(4):
            j = q * 4 + k

            @pl.when(j < nch)
            def _():
                pltpu.make_async_copy(h_hbm.at[sidx.at[j]], rows[k],
                                      gsems[k]).wait()
                pltpu.async_copy(rows[k], acc.at[didx.at[j]], ssems[k],
                                 add=True)

                @pl.when(j + 2 < nch)
                def _():
                    k2 = (k + 2) % 4

                    @pl.when(j >= 2)
                    def _():
                        pltpu.make_async_copy(rows[k2], acc.at[didx.at[0]],
                                              ssems[k2]).wait()

                    pltpu.async_copy(h_hbm.at[sidx.at[j + 2]], rows[k2],
                                     gsems[k2])
        return 0

    lax.fori_loop(0, (nch + 3) // 4, group, 0, unroll=False)
    for k in range(4):
        pltpu.make_async_copy(rows[k], acc.at[didx.at[0]], ssems[k]).wait()


_PROP_SCRATCH = [
    pltpu.VMEM((NCH, CH), jnp.int32),
    pltpu.VMEM((NCH, CH), jnp.int32),
    pltpu.VMEM((CH, HF), jnp.float32),
    pltpu.VMEM((CH, HF), jnp.float32),
    pltpu.VMEM((CH, HF), jnp.float32),
    pltpu.VMEM((CH, HF), jnp.float32),
    pltpu.VMEM_SHARED((NPAD, HF), jnp.float32),
] + [pltpu.SemaphoreType.DMA] * 8


@functools.partial(
    pl.kernel,
    out_type=jax.ShapeDtypeStruct((NC, NPAD, HF), jnp.float32),
    mesh=_mesh,
    compiler_params=_sc_params,
    scratch_types=_PROP_SCRATCH,
)
def _prop(h_hbm, src_hbm, dst_hbm, out_hbm, sidx, didx, r0, r1, r2, r3, acc,
          g0, g1, g2, g3, s0, s1, s2, s3):
    c = lax.axis_index("c")
    s = lax.axis_index("s")
    w = s * NC + c
    base = s * ROWS_PER_TILE

    cs = pltpu.async_copy(src_hbm.at[w], sidx, g1)
    cd = pltpu.async_copy(dst_hbm.at[w], didx, g2)
    _zero_acc(r0, acc, base, g0)
    cs.wait()
    cd.wait()

    off = jnp.zeros((16,), jnp.int32) + c * NPAD

    def adj_body(r, _):
        for k in range(CH // 16):
            sl = pl.ds(k * 16, 16)
            sidx[r, sl] = sidx[r, sl] + off
        return 0

    lax.fori_loop(0, NCH, adj_body, 0, unroll=False)
    plsc.subcore_barrier()

    _ring_loop(h_hbm, acc, sidx, didx, (r0, r1, r2, r3),
               (g0, g1, g2, g3), (s0, s1, s2, s3), NCH)
    plsc.subcore_barrier()

    pltpu.sync_copy(acc.at[pl.ds(base, ROWS_PER_TILE)],
                    out_hbm.at[c, pl.ds(base, ROWS_PER_TILE)])


NCH2 = NCH * 2

_PROP2_SCRATCH = [
    pltpu.VMEM((NCH2, CH), jnp.int32),
    pltpu.VMEM((NCH2, CH), jnp.int32),
    pltpu.VMEM((CH, HF), jnp.float32),
    pltpu.VMEM((CH, HF), jnp.float32),
    pltpu.VMEM((CH, HF), jnp.float32),
    pltpu.VMEM((CH, HF), jnp.float32),
    pltpu.VMEM_SHARED((NPAD, HF), jnp.float32),
] + [pltpu.SemaphoreType.DMA] * 8


@functools.partial(
    pl.kernel,
    out_type=jax.ShapeDtypeStruct((NC, NPAD, HF), jnp.float32),
    mesh=_mesh,
    compiler_params=_sc_params,
    scratch_types=_PROP2_SCRATCH,
)
def _prop_dual(h_hbm, src_hbm, dst_hbm, out_hbm, sidx, didx, r0, r1, r2, r3,
               acc, g0, g1, g2, g3, s0, s1, s2, s3):
    c = lax.axis_index("c")
    s = lax.axis_index("s")
    base = s * ROWS_PER_TILE

    cs = pltpu.async_copy(src_hbm.at[s], sidx, g1)
    cd = pltpu.async_copy(dst_hbm.at[s], didx, g2)
    _zero_acc(r0, acc, base, g0)
    cs.wait()
    cd.wait()

    off = jnp.zeros((16,), jnp.int32) + c * NPAD

    def adj_body(r, _):
        for k in range(CH // 16):
            sl = pl.ds(k * 16, 16)
            sidx[r, sl] = sidx[r, sl] + off
        return 0

    lax.fori_loop(0, NCH2, adj_body, 0, unroll=False)
    plsc.subcore_barrier()

    _ring_loop(h_hbm, acc, sidx, didx, (r0, r1, r2, r3),
               (g0, g1, g2, g3), (s0, s1, s2, s3), NCH2)
    plsc.subcore_barrier()

    pltpu.sync_copy(acc.at[pl.ds(base, ROWS_PER_TILE)],
                    out_hbm.at[c, pl.ds(base, ROWS_PER_TILE)])


def _norm_from(deg_ref):
    d = deg_ref[0, :, 0:1] + deg_ref[1, :, 0:1]
    return lax.rsqrt(jnp.maximum(d, 1.0))


def _mm(a, w):
    return jnp.dot(a, w, preferred_element_type=jnp.float32,
                   precision=lax.Precision.HIGHEST)


def _tc_pre_body(x_ref, degs_ref, o_ref):
    h = x_ref[...] * _norm_from(degs_ref)
    o_ref[0] = h[:, :HF]
    o_ref[1] = h[:, HF:]


def _tc_layer1_body(agg_ref, degd_ref, degs_ref, w_ref, b_ref, o_ref):
    nd = _norm_from(degd_ref)
    y = (_mm(agg_ref[0] * nd, w_ref[:HF, :])
         + _mm(agg_ref[1] * nd, w_ref[HF:, :]) + b_ref[...])
    y = jnp.maximum(y, 0.0) * _norm_from(degs_ref)
    o_ref[0] = y[:, :HF]
    o_ref[1] = y[:, HF:]


def _tc_layer2_body(agg_ref, degd_ref, degs_ref, w_ref, b_ref, w3_ref, o_ref):
    nd = _norm_from(degd_ref)
    y = (_mm(agg_ref[0] * nd, w_ref[:HF, :])
         + _mm(agg_ref[1] * nd, w_ref[HF:, :]) + b_ref[...])
    y = jnp.maximum(y, 0.0)
    t = _mm(y, w3_ref[...]) * _norm_from(degs_ref)
    o_ref[0] = t
    o_ref[1] = t


def _tc_fin_body(agg_ref, degd_ref, b_ref, o_ref):
    a = agg_ref[0] + agg_ref[1]
    o_ref[...] = a * _norm_from(degd_ref) + b_ref[...]


_R = 1024
_GRID = NPAD // _R


def _deg_spec():
    return pl.BlockSpec((NC, _R, DEG_W), lambda i: (0, i, 0))


def _agg_spec():
    return pl.BlockSpec((NC, _R, HF), lambda i: (0, i, 0))


def _full_spec(r, c):
    return pl.BlockSpec((r, c), lambda i: (0, 0))


def _row_spec(c):
    return pl.BlockSpec((_R, c), lambda i: (i, 0))


_stacked_out = jax.ShapeDtypeStruct((NC, NPAD, HF), jnp.float32)

_tc_pre = pl.pallas_call(
    _tc_pre_body,
    grid=(_GRID,),
    in_specs=[_row_spec(IN_FEATS), _deg_spec()],
    out_specs=_agg_spec(),
    out_shape=_stacked_out,
)

_tc_layer1 = pl.pallas_call(
    _tc_layer1_body,
    grid=(_GRID,),
    in_specs=[
        _agg_spec(), _deg_spec(), _deg_spec(),
        _full_spec(IN_FEATS, H_FEATS), _full_spec(1, H_FEATS),
    ],
    out_specs=_agg_spec(),
    out_shape=_stacked_out,
)

_tc_layer2 = pl.pallas_call(
    _tc_layer2_body,
    grid=(_GRID,),
    in_specs=[
        _agg_spec(), _deg_spec(), _deg_spec(),
        _full_spec(H_FEATS, H_FEATS), _full_spec(1, H_FEATS),
        _full_spec(H_FEATS, NUM_CLASSES),
    ],
    out_specs=pl.BlockSpec((NC, _R, NUM_CLASSES), lambda i: (0, i, 0)),
    out_shape=jax.ShapeDtypeStruct((NC, NPAD, NUM_CLASSES), jnp.float32),
)

_tc_fin = pl.pallas_call(
    _tc_fin_body,
    grid=(_GRID,),
    in_specs=[
        pl.BlockSpec((NC, _R, NUM_CLASSES), lambda i: (0, i, 0)),
        _deg_spec(),
        _full_spec(1, NUM_CLASSES),
    ],
    out_specs=_row_spec(NUM_CLASSES),
    out_shape=jax.ShapeDtypeStruct((NPAD, NUM_CLASSES), jnp.float32),
)


def kernel(in_feat, edge_index, W1, b1, W2, b2, W3, b3):
    src = edge_index[0].astype(jnp.int32)
    dst = edge_index[1].astype(jnp.int32)
    pad = jnp.full((EPAD - N_EDGES,), DUMMY, jnp.int32)
    srcf = jnp.concatenate([src, pad])
    dstf = jnp.concatenate([dst, pad])
    src32 = srcf.reshape(NW, NCH, CH)
    dst32 = dstf.reshape(NW, NCH, CH)
    src16 = srcf.reshape(NS, NCH2, CH)
    dst16 = dstf.reshape(NS, NCH2, CH)
    x_pad = jnp.zeros((NPAD, IN_FEATS), jnp.float32).at[:N_NODES].set(in_feat)

    degs, degd = _deg_kernel(src32, dst32)

    h0 = _tc_pre(x_pad, degs).reshape(NC * NPAD, HF)
    a1 = _prop_dual(h0, src16, dst16)
    h1 = _tc_layer1(a1, degd, degs, W1, b1.reshape(1, -1)).reshape(NC * NPAD, HF)
    a2 = _prop_dual(h1, src16, dst16)
    t2 = _tc_layer2(a2, degd, degs, W2, b2.reshape(1, -1), W3)
    a3 = _prop(t2.reshape(NC * NPAD, NUM_CLASSES), src32, dst32)
    out = _tc_fin(a3, degd, b3.reshape(1, -1))
    return out[:N_NODES]


# speedup vs baseline: 1.5243x; 1.5243x over previous
import functools

import jax
import jax.numpy as jnp
from jax import lax
from jax.experimental import pallas as pl
from jax.experimental.pallas import tpu as pltpu
from jax.experimental.pallas import tpu_sc as plsc

N_NODES = 10000
N_EDGES = 320000
IN_FEATS = 128
H_FEATS = 128
NUM_CLASSES = 64

NPAD = 10240
DUMMY = NPAD - 1
NC = 2
NS = 16
NW = NC * NS
CH = 128
NCH = 79
EPAD = NW * NCH * CH
ROWS_PER_TILE = NPAD // NS
HF = 64
DEG_W = 16

_mesh = plsc.VectorSubcoreMesh(core_axis_name="c", subcore_axis_name="s")
_sc_params = pltpu.CompilerParams(use_tc_tiling_on_sc=False)


def _zero_fill(ref, nrows, width):
    zv = jnp.zeros((16,), jnp.float32)

    def body(i, _):
        for k in range(width // 16):
            ref[i, pl.ds(k * 16, 16)] = zv
        return 0

    lax.fori_loop(0, nrows, body, 0, unroll=False)


def _ones_fill(ref, nrows, width):
    ov = jnp.ones((16,), jnp.float32)

    def body(i, _):
        for k in range(width // 16):
            ref[i, pl.ds(k * 16, 16)] = ov
        return 0

    lax.fori_loop(0, nrows, body, 0, unroll=False)


@functools.partial(
    pl.kernel,
    out_type=(
        jax.ShapeDtypeStruct((NC, NPAD, DEG_W), jnp.float32),
        jax.ShapeDtypeStruct((NC, NPAD, DEG_W), jnp.float32),
    ),
    mesh=_mesh,
    compiler_params=_sc_params,
    scratch_types=[
        pltpu.VMEM((NCH, CH), jnp.int32),
        pltpu.VMEM((NCH, CH), jnp.int32),
        pltpu.VMEM((CH, DEG_W), jnp.float32),
        pltpu.VMEM_SHARED((NPAD, DEG_W), jnp.float32),
        pltpu.VMEM_SHARED((NPAD, DEG_W), jnp.float32),
        pltpu.SemaphoreType.DMA,
        pltpu.SemaphoreType.DMA,
    ],
)
def _deg_kernel(src_hbm, dst_hbm, outs_hbm, outd_hbm, sidx, didx, buf, accs,
                accd, sems, semd):
    c = lax.axis_index("c")
    s = lax.axis_index("s")
    w = s * NC + c
    base = s * ROWS_PER_TILE
    _zero_fill(buf, CH, DEG_W)
    for r in range(ROWS_PER_TILE // CH):
        pltpu.sync_copy(buf, accs.at[pl.ds(base + r * CH, CH)])
        pltpu.sync_copy(buf, accd.at[pl.ds(base + r * CH, CH)])
    _ones_fill(buf, CH, DEG_W)
    plsc.subcore_barrier()

    pltpu.sync_copy(src_hbm.at[w], sidx)
    pltpu.sync_copy(dst_hbm.at[w], didx)

    pltpu.async_copy(buf, accs.at[sidx.at[0]], sems, add=True)
    pltpu.async_copy(buf, accd.at[didx.at[0]], semd, add=True)

    def body(j, _):
        pltpu.make_async_copy(buf, accs.at[sidx.at[j - 1]], sems).wait()
        pltpu.make_async_copy(buf, accd.at[didx.at[j - 1]], semd).wait()
        pltpu.async_copy(buf, accs.at[sidx.at[j]], sems, add=True)
        pltpu.async_copy(buf, accd.at[didx.at[j]], semd, add=True)
        return 0

    lax.fori_loop(1, NCH, body, 0, unroll=False)
    pltpu.make_async_copy(buf, accs.at[sidx.at[NCH - 1]], sems).wait()
    pltpu.make_async_copy(buf, accd.at[didx.at[NCH - 1]], semd).wait()
    plsc.subcore_barrier()

    pltpu.sync_copy(accs.at[pl.ds(base, ROWS_PER_TILE)],
                    outs_hbm.at[c, pl.ds(base, ROWS_PER_TILE)])
    pltpu.sync_copy(accd.at[pl.ds(base, ROWS_PER_TILE)],
                    outd_hbm.at[c, pl.ds(base, ROWS_PER_TILE)])


def _zero_acc(zbuf, acc, base, zsem):
    _zero_fill(zbuf, CH, HF)
    ds = [pltpu.async_copy(zbuf, acc.at[pl.ds(base + r * CH, CH)], zsem)
          for r in range(ROWS_PER_TILE // CH)]
    for d in ds:
        d.wait()


def _ring_loop(h_hbm, acc, sidx, didx, rows, gsems, ssems, nch):
    pltpu.async_copy(h_hbm.at[sidx.at[0]], rows[0], gsems[0])
    pltpu.async_copy(h_hbm.at[sidx.at[1]], rows[1], gsems[1])

    def group(q, _):
        for k in range(4):
            j = q * 4 + k

            @pl.when(j < nch)
            def _():
                pltpu.make_async_copy(h_hbm.at[sidx.at[j]], rows[k],
                                      gsems[k]).wait()
                pltpu.async_copy(rows[k], acc.at[didx.at[j]], ssems[k],
                                 add=True)

                @pl.when(j + 2 < nch)
                def _():
                    k2 = (k + 2) % 4

                    @pl.when(j >= 2)
                    def _():
                        pltpu.make_async_copy(rows[k2], acc.at[didx.at[0]],
                                              ssems[k2]).wait()

                    pltpu.async_copy(h_hbm.at[sidx.at[j + 2]], rows[k2],
                                     gsems[k2])
        return 0

    lax.fori_loop(0, (nch + 3) // 4, group, 0, unroll=False)
    for k in range(4):
        pltpu.make_async_copy(rows[k], acc.at[didx.at[0]], ssems[k]).wait()


_PROP_SCRATCH = [
    pltpu.VMEM((NCH, CH), jnp.int32),
    pltpu.VMEM((NCH, CH), jnp.int32),
    pltpu.VMEM((CH, HF), jnp.float32),
    pltpu.VMEM((CH, HF), jnp.float32),
    pltpu.VMEM((CH, HF), jnp.float32),
    pltpu.VMEM((CH, HF), jnp.float32),
    pltpu.VMEM_SHARED((NPAD, HF), jnp.float32),
] + [pltpu.SemaphoreType.DMA] * 8


@functools.partial(
    pl.kernel,
    out_type=jax.ShapeDtypeStruct((NC, NPAD, HF), jnp.float32),
    mesh=_mesh,
    compiler_params=_sc_params,
    scratch_types=_PROP_SCRATCH,
)
def _prop(h_hbm, src_hbm, dst_hbm, out_hbm, sidx, didx, r0, r1, r2, r3, acc,
          g0, g1, g2, g3, s0, s1, s2, s3):
    c = lax.axis_index("c")
    s = lax.axis_index("s")
    w = s * NC + c
    base = s * ROWS_PER_TILE

    cs = pltpu.async_copy(src_hbm.at[w], sidx, g1)
    cd = pltpu.async_copy(dst_hbm.at[w], didx, g2)
    _zero_acc(r0, acc, base, g0)
    cs.wait()
    cd.wait()
    plsc.subcore_barrier()

    _ring_loop(h_hbm, acc, sidx, didx, (r0, r1, r2, r3),
               (g0, g1, g2, g3), (s0, s1, s2, s3), NCH)
    plsc.subcore_barrier()

    pltpu.sync_copy(acc.at[pl.ds(base, ROWS_PER_TILE)],
                    out_hbm.at[c, pl.ds(base, ROWS_PER_TILE)])


NCH2 = NCH * 2

_PROP2_SCRATCH = [
    pltpu.VMEM((NCH2, CH), jnp.int32),
    pltpu.VMEM((NCH2, CH), jnp.int32),
    pltpu.VMEM((CH, HF), jnp.float32),
    pltpu.VMEM((CH, HF), jnp.float32),
    pltpu.VMEM((CH, HF), jnp.float32),
    pltpu.VMEM((CH, HF), jnp.float32),
    pltpu.VMEM_SHARED((NPAD, HF), jnp.float32),
] + [pltpu.SemaphoreType.DMA] * 8


@functools.partial(
    pl.kernel,
    out_type=jax.ShapeDtypeStruct((NC, NPAD, HF), jnp.float32),
    mesh=_mesh,
    compiler_params=_sc_params,
    scratch_types=_PROP2_SCRATCH,
)
def _prop_dual(h_hbm, src_hbm, dst_hbm, out_hbm, sidx, didx, r0, r1, r2, r3,
               acc, g0, g1, g2, g3, s0, s1, s2, s3):
    c = lax.axis_index("c")
    s = lax.axis_index("s")
    base = s * ROWS_PER_TILE

    cs = pltpu.async_copy(src_hbm.at[s], sidx, g1)
    cd = pltpu.async_copy(dst_hbm.at[s], didx, g2)
    _zero_acc(r0, acc, base, g0)
    cs.wait()
    cd.wait()

    off = jnp.zeros((16,), jnp.int32) + c * NPAD

    def adj_body(r, _):
        for k in range(CH // 16):
            sl = pl.ds(k * 16, 16)
            sidx[r, sl] = sidx[r, sl] + off
        return 0

    lax.fori_loop(0, NCH2, adj_body, 0, unroll=False)
    plsc.subcore_barrier()

    _ring_loop(h_hbm, acc, sidx, didx, (r0, r1, r2, r3),
               (g0, g1, g2, g3), (s0, s1, s2, s3), NCH2)
    plsc.subcore_barrier()

    pltpu.sync_copy(acc.at[pl.ds(base, ROWS_PER_TILE)],
                    out_hbm.at[c, pl.ds(base, ROWS_PER_TILE)])


def _norm_from(deg_ref):
    d = deg_ref[0, :, 0:1] + deg_ref[1, :, 0:1]
    return lax.rsqrt(jnp.maximum(d, 1.0))


def _mm(a, w):
    return jnp.dot(a, w, preferred_element_type=jnp.float32,
                   precision=lax.Precision.HIGHEST)


def _tc_pre_body(x_ref, degs_ref, o_ref):
    h = x_ref[...] * _norm_from(degs_ref)
    o_ref[0] = h[:, :HF]
    o_ref[1] = h[:, HF:]


def _tc_layer1_body(agg_ref, degd_ref, degs_ref, w_ref, b_ref, o_ref):
    nd = _norm_from(degd_ref)
    y = (_mm(agg_ref[0] * nd, w_ref[:HF, :])
         + _mm(agg_ref[1] * nd, w_ref[HF:, :]) + b_ref[...])
    y = jnp.maximum(y, 0.0) * _norm_from(degs_ref)
    o_ref[0] = y[:, :HF]
    o_ref[1] = y[:, HF:]


def _tc_layer2_body(agg_ref, degd_ref, degs_ref, w_ref, b_ref, w3_ref, o_ref):
    nd = _norm_from(degd_ref)
    y = (_mm(agg_ref[0] * nd, w_ref[:HF, :])
         + _mm(agg_ref[1] * nd, w_ref[HF:, :]) + b_ref[...])
    y = jnp.maximum(y, 0.0)
    o_ref[...] = _mm(y, w3_ref[...]) * _norm_from(degs_ref)


def _tc_fin_body(agg_ref, degd_ref, b_ref, o_ref):
    a = agg_ref[0] + agg_ref[1]
    o_ref[...] = a * _norm_from(degd_ref) + b_ref[...]


_R = 1024
_GRID = NPAD // _R


def _deg_spec():
    return pl.BlockSpec((NC, _R, DEG_W), lambda i: (0, i, 0))


def _agg_spec():
    return pl.BlockSpec((NC, _R, HF), lambda i: (0, i, 0))


def _full_spec(r, c):
    return pl.BlockSpec((r, c), lambda i: (0, 0))


def _row_spec(c):
    return pl.BlockSpec((_R, c), lambda i: (i, 0))


_stacked_out = jax.ShapeDtypeStruct((NC, NPAD, HF), jnp.float32)

_tc_pre = pl.pallas_call(
    _tc_pre_body,
    grid=(_GRID,),
    in_specs=[_row_spec(IN_FEATS), _deg_spec()],
    out_specs=_agg_spec(),
    out_shape=_stacked_out,
)

_tc_layer1 = pl.pallas_call(
    _tc_layer1_body,
    grid=(_GRID,),
    in_specs=[
        _agg_spec(), _deg_spec(), _deg_spec(),
        _full_spec(IN_FEATS, H_FEATS), _full_spec(1, H_FEATS),
    ],
    out_specs=_agg_spec(),
    out_shape=_stacked_out,
)

_tc_layer2 = pl.pallas_call(
    _tc_layer2_body,
    grid=(_GRID,),
    in_specs=[
        _agg_spec(), _deg_spec(), _deg_spec(),
        _full_spec(H_FEATS, H_FEATS), _full_spec(1, H_FEATS),
        _full_spec(H_FEATS, NUM_CLASSES),
    ],
    out_specs=_row_spec(NUM_CLASSES),
    out_shape=jax.ShapeDtypeStruct((NPAD, NUM_CLASSES), jnp.float32),
)

_tc_fin = pl.pallas_call(
    _tc_fin_body,
    grid=(_GRID,),
    in_specs=[
        pl.BlockSpec((NC, _R, NUM_CLASSES), lambda i: (0, i, 0)),
        _deg_spec(),
        _full_spec(1, NUM_CLASSES),
    ],
    out_specs=_row_spec(NUM_CLASSES),
    out_shape=jax.ShapeDtypeStruct((NPAD, NUM_CLASSES), jnp.float32),
)


def kernel(in_feat, edge_index, W1, b1, W2, b2, W3, b3):
    src = edge_index[0].astype(jnp.int32)
    dst = edge_index[1].astype(jnp.int32)
    pad = N_NODES + (jnp.arange(EPAD - N_EDGES, dtype=jnp.int32)
                     % (NPAD - N_NODES))
    srcf = jnp.concatenate([src, pad])
    dstf = jnp.concatenate([dst, pad])
    src32 = srcf.reshape(NW, NCH, CH)
    dst32 = dstf.reshape(NW, NCH, CH)
    src16 = srcf.reshape(NS, NCH2, CH)
    dst16 = dstf.reshape(NS, NCH2, CH)
    x_pad = jnp.zeros((NPAD, IN_FEATS), jnp.float32).at[:N_NODES].set(in_feat)

    degs, degd = _deg_kernel(src32, dst32)

    h0 = _tc_pre(x_pad, degs).reshape(NC * NPAD, HF)
    a1 = _prop_dual(h0, src16, dst16)
    h1 = _tc_layer1(a1, degd, degs, W1, b1.reshape(1, -1)).reshape(NC * NPAD, HF)
    a2 = _prop_dual(h1, src16, dst16)
    t2 = _tc_layer2(a2, degd, degs, W2, b2.reshape(1, -1), W3)
    a3 = _prop(t2, src32, dst32)
    out = _tc_fin(a3, degd, b3.reshape(1, -1))
    return out[:N_NODES]
